# baseline (speedup 1.0000x reference)
import functools

import jax
import jax.numpy as jnp
from jax import lax
from jax.experimental import pallas as pl
from jax.experimental.pallas import tpu as pltpu
from jax.experimental.pallas import tpu_sc as plsc


_TC_ROWS = 6144
_TC_CHUNK = 1024
_SC_CHUNK = 32


def _tc_body(src_hbm, dst_hbm, buf, in_sems, out_sems):
    n = dst_hbm.shape[0] // _TC_CHUNK

    def in_copy(i):
        return pltpu.make_async_copy(
            src_hbm.at[pl.ds(i * _TC_CHUNK, _TC_CHUNK)], buf.at[i],
            in_sems.at[i])

    def out_copy(i):
        return pltpu.make_async_copy(
            buf.at[i], dst_hbm.at[pl.ds(i * _TC_CHUNK, _TC_CHUNK)],
            out_sems.at[i])

    for i in range(n):
        in_copy(i).start()
    for i in range(n):
        in_copy(i).wait()
        out_copy(i).start()
    for i in range(n):
        out_copy(i).wait()


def _sc_body(pe_hbm, out_hbm, buf, in_sems, out_sems):
    nw = 32
    row_off = pe_hbm.shape[0] - out_hbm.shape[0]
    rows_per_w = out_hbm.shape[0] // nw
    n = rows_per_w // _SC_CHUNK
    wid = lax.axis_index("s") * 2 + lax.axis_index("c")
    base = wid * rows_per_w

    def in_copy(i, slot):
        return pltpu.make_async_copy(
            pe_hbm.at[pl.ds(row_off + base + i * _SC_CHUNK, _SC_CHUNK)],
            buf.at[slot], in_sems.at[slot])

    def out_copy(i, slot):
        return pltpu.make_async_copy(
            buf.at[slot],
            out_hbm.at[pl.ds(base + i * _SC_CHUNK, _SC_CHUNK)],
            out_sems.at[slot])

    in_copy(0, 0).start()
    for i in range(n):
        slot = i % 2
        in_copy(i, slot).wait()
        out_copy(i, slot).start()
        if i + 1 < n:
            nslot = (i + 1) % 2
            if i - 1 >= 0:
                out_copy(i - 1, nslot).wait()
            in_copy(i + 1, nslot).start()
    if n >= 2:
        out_copy(n - 2, (n - 2) % 2).wait()
    out_copy(n - 1, (n - 1) % 2).wait()


def kernel(x, pe_weight):
    seq_len = x.shape[1]
    hidden = pe_weight.shape[1]
    n_tc = _TC_ROWS // _TC_CHUNK

    tc_part = pl.pallas_call(
        _tc_body,
        out_shape=jax.ShapeDtypeStruct((_TC_ROWS, hidden), pe_weight.dtype),
        in_specs=[pl.BlockSpec(memory_space=pl.ANY)],
        out_specs=pl.BlockSpec(memory_space=pl.ANY),
        scratch_shapes=[
            pltpu.VMEM((n_tc, _TC_CHUNK, hidden), pe_weight.dtype),
            pltpu.SemaphoreType.DMA((n_tc,)),
            pltpu.SemaphoreType.DMA((n_tc,)),
        ],
    )(pe_weight)

    sc_rows = seq_len - _TC_ROWS
    sc_call = functools.partial(
        pl.kernel,
        mesh=plsc.VectorSubcoreMesh(core_axis_name="c", subcore_axis_name="s"),
        out_type=jax.ShapeDtypeStruct((sc_rows, hidden), pe_weight.dtype),
        scratch_types=[
            pltpu.VMEM((2, _SC_CHUNK, hidden), pe_weight.dtype),
            pltpu.SemaphoreType.DMA((2,)),
            pltpu.SemaphoreType.DMA((2,)),
        ],
    )(_sc_body)
    sc_part = sc_call(pe_weight)

    return jnp.concatenate([tc_part, sc_part], axis=0)[None]


# speedup vs baseline: 2.8236x; 2.8236x over previous
import jax
import jax.numpy as jnp
from jax.experimental import pallas as pl
from jax.experimental.pallas import tpu as pltpu


_CHUNK_ROWS = 512


def _copy_body(src_hbm, dst_hbm, buf, in_sems, out_sems):
    rows = src_hbm.shape[0]
    n = rows // _CHUNK_ROWS

    def in_copy(i):
        return pltpu.make_async_copy(
            src_hbm.at[pl.ds(i * _CHUNK_ROWS, _CHUNK_ROWS)], buf.at[i],
            in_sems.at[i])

    def out_copy(i):
        return pltpu.make_async_copy(
            buf.at[i], dst_hbm.at[0, pl.ds(i * _CHUNK_ROWS, _CHUNK_ROWS)],
            out_sems.at[i])

    for i in range(n):
        in_copy(i).start()
    for i in range(n):
        in_copy(i).wait()
        out_copy(i).start()
    for i in range(n):
        out_copy(i).wait()


def kernel(x, pe_weight):
    seq_len = x.shape[1]
    hidden = pe_weight.shape[1]
    n = seq_len // _CHUNK_ROWS
    return pl.pallas_call(
        _copy_body,
        out_shape=jax.ShapeDtypeStruct((1, seq_len, hidden), pe_weight.dtype),
        in_specs=[pl.BlockSpec(memory_space=pl.ANY)],
        out_specs=pl.BlockSpec(memory_space=pl.ANY),
        scratch_shapes=[
            pltpu.VMEM((n, _CHUNK_ROWS, hidden), pe_weight.dtype),
            pltpu.SemaphoreType.DMA((n,)),
            pltpu.SemaphoreType.DMA((n,)),
        ],
    )(pe_weight)


# speedup vs baseline: 2.8954x; 1.0254x over previous
import jax
import jax.numpy as jnp
from jax.experimental import pallas as pl
from jax.experimental.pallas import tpu as pltpu


_CHUNK_ROWS = 2048


def _copy_body(src_hbm, dst_hbm, buf, in_sems, out_sems):
    rows = src_hbm.shape[0]
    n = rows // _CHUNK_ROWS

    def in_copy(i):
        return pltpu.make_async_copy(
            src_hbm.at[pl.ds(i * _CHUNK_ROWS, _CHUNK_ROWS)], buf.at[i],
            in_sems.at[i])

    def out_copy(i):
        return pltpu.make_async_copy(
            buf.at[i], dst_hbm.at[0, pl.ds(i * _CHUNK_ROWS, _CHUNK_ROWS)],
            out_sems.at[i])

    for i in range(n):
        in_copy(i).start()
    for i in range(n):
        in_copy(i).wait()
        out_copy(i).start()
    for i in range(n):
        out_copy(i).wait()


def kernel(x, pe_weight):
    seq_len = x.shape[1]
    hidden = pe_weight.shape[1]
    n = seq_len // _CHUNK_ROWS
    return pl.pallas_call(
        _copy_body,
        out_shape=jax.ShapeDtypeStruct((1, seq_len, hidden), pe_weight.dtype),
        in_specs=[pl.BlockSpec(memory_space=pl.ANY)],
        out_specs=pl.BlockSpec(memory_space=pl.ANY),
        scratch_shapes=[
            pltpu.VMEM((n, _CHUNK_ROWS, hidden), pe_weight.dtype),
            pltpu.SemaphoreType.DMA((n,)),
            pltpu.SemaphoreType.DMA((n,)),
        ],
    )(pe_weight)


# speedup vs baseline: 2.9206x; 1.0087x over previous
import jax
import jax.numpy as jnp
from jax.experimental import pallas as pl
from jax.experimental.pallas import tpu as pltpu


_CHUNK_ROWS = 4096


def _copy_body(src_hbm, dst_hbm, buf, in_sems, out_sems):
    rows = src_hbm.shape[0]
    n = rows // _CHUNK_ROWS

    def in_copy(i):
        return pltpu.make_async_copy(
            src_hbm.at[pl.ds(i * _CHUNK_ROWS, _CHUNK_ROWS)], buf.at[i],
            in_sems.at[i])

    def out_copy(i):
        return pltpu.make_async_copy(
            buf.at[i], dst_hbm.at[0, pl.ds(i * _CHUNK_ROWS, _CHUNK_ROWS)],
            out_sems.at[i])

    for i in range(n):
        in_copy(i).start()
    for i in range(n):
        in_copy(i).wait()
        out_copy(i).start()
    for i in range(n):
        out_copy(i).wait()


def kernel(x, pe_weight):
    seq_len = x.shape[1]
    hidden = pe_weight.shape[1]
    n = seq_len // _CHUNK_ROWS
    return pl.pallas_call(
        _copy_body,
        out_shape=jax.ShapeDtypeStruct((1, seq_len, hidden), pe_weight.dtype),
        in_specs=[pl.BlockSpec(memory_space=pl.ANY)],
        out_specs=pl.BlockSpec(memory_space=pl.ANY),
        scratch_shapes=[
            pltpu.VMEM((n, _CHUNK_ROWS, hidden), pe_weight.dtype),
            pltpu.SemaphoreType.DMA((n,)),
            pltpu.SemaphoreType.DMA((n,)),
        ],
    )(pe_weight)
